# baseline (speedup 1.0000x reference)
import functools

import jax
import jax.numpy as jnp
import numpy as np
from jax.experimental import pallas as pl
from jax.experimental.pallas import tpu as pltpu

N_FFT = 1024
HOP = 256
WIN = 1024
EPS = 1e-11
NFREQ = N_FFT // 2 + 1
PAD = (WIN - HOP) // 2
BQ = 256
HALO = 128


def _idft_matrix() -> np.ndarray:
    k = np.arange(NFREQ, dtype=np.float64)
    n = np.arange(N_FFT, dtype=np.float64)
    coef = np.full(NFREQ, 2.0)
    coef[0] = 1.0
    coef[NFREQ - 1] = 1.0
    m = (coef[None, :] * np.cos(2.0 * np.pi * np.outer(n, k) / N_FFT)) / N_FFT
    return m.astype(np.float32)


def _istft_kernel(t_total, main_ref, hi_ref, m_ref, w_ref, out_ref, c_ref):
    k = pl.program_id(1)

    @pl.when(k == 0)
    def _init():
        c_ref[...] = jnp.zeros_like(c_ref)

    a = m_ref[...] * w_ref[...]
    wsq1 = w_ref[...] * w_ref[...]

    q_idx = k * BQ + jax.lax.broadcasted_iota(jnp.int32, (1, BQ), 1)
    masks = {}
    for d in range(-2, 3):
        t = q_idx - d
        masks[d] = jnp.logical_and(t >= 0, t < t_total).astype(jnp.float32)

    env_top = (masks[-1] * wsq1[128 + 0:128 + 128] + masks[0] * wsq1[384:512]
               + masks[1] * wsq1[640:768] + masks[2] * wsq1[896:1024])
    env_bot = (masks[-2] * wsq1[0:128] + masks[-1] * wsq1[256:384]
               + masks[0] * wsq1[512:640] + masks[1] * wsq1[768:896])
    inv_env = 1.0 / (jnp.concatenate([env_top, env_bot], axis=0) + EPS)

    x = jnp.concatenate([c_ref[...], main_ref[0], hi_ref[0]], axis=1)
    acc = jnp.zeros((HOP, BQ), dtype=jnp.float32)
    for d in (-1, 0, 1):
        xs = x[:, 128 - d:384 - d]
        ad = a[384 + 256 * d:640 + 256 * d, :]
        acc = acc + masks[d] * jax.lax.dot_general(
            ad, xs, (((1,), (0,)), ((), ())),
            preferred_element_type=jnp.float32)
    top = acc[0:128, :] + masks[2] * jax.lax.dot_general(
        a[896:1024, :], x[:, 126:382], (((1,), (0,)), ((), ())),
        preferred_element_type=jnp.float32)
    bot = acc[128:256, :] + masks[-2] * jax.lax.dot_general(
        a[0:128, :], x[:, 130:386], (((1,), (0,)), ((), ())),
        preferred_element_type=jnp.float32)
    y = jnp.concatenate([top, bot], axis=0) * inv_env
    out_ref[0] = y.T

    c_ref[...] = main_ref[0, :, BQ - HALO:]


@jax.jit
def kernel(spec, window):
    b, nfreq, t = spec.shape
    n_chunks = t // BQ
    t_halo_blocks = t // HALO

    m = jnp.asarray(_idft_matrix())
    w2d = window.reshape(WIN, 1)

    out = pl.pallas_call(
        functools.partial(_istft_kernel, t),
        grid=(b, n_chunks),
        in_specs=[
            pl.BlockSpec((1, nfreq, BQ), lambda bi, k: (bi, 0, k)),
            pl.BlockSpec((1, nfreq, HALO),
                         lambda bi, k: (bi, 0,
                                        jnp.clip(2 * k + 2, 0,
                                                 t // HALO - 1))),
            pl.BlockSpec((WIN, NFREQ), lambda bi, k: (0, 0)),
            pl.BlockSpec((WIN, 1), lambda bi, k: (0, 0)),
        ],
        out_specs=pl.BlockSpec((1, BQ, HOP), lambda bi, k: (bi, k, 0)),
        out_shape=jax.ShapeDtypeStruct((b, t, HOP), jnp.float32),
        scratch_shapes=[
            pltpu.VMEM((nfreq, HALO), jnp.float32),
        ],
    )(spec, spec, m, w2d)

    return out.reshape(b, t * HOP)


# speedup vs baseline: 1.4660x; 1.4660x over previous
import functools

import jax
import jax.numpy as jnp
import numpy as np
from jax.experimental import pallas as pl
from jax.experimental.pallas import tpu as pltpu

N_FFT = 1024
HOP = 256
WIN = 1024
EPS = 1e-11
NFREQ = N_FFT // 2 + 1
PAD = (WIN - HOP) // 2
BQ = 256
HALO = 128


def _idft_matrix() -> np.ndarray:
    k = np.arange(NFREQ, dtype=np.float64)
    n = np.arange(N_FFT, dtype=np.float64)
    coef = np.full(NFREQ, 2.0)
    coef[0] = 1.0
    coef[NFREQ - 1] = 1.0
    m = (coef[None, :] * np.cos(2.0 * np.pi * np.outer(n, k) / N_FFT)) / N_FFT
    return m.astype(np.float32)


def _istft_kernel(t_total, main_ref, a_ref, w_ref, out_ref, s1_ref, s2_ref):
    k = pl.program_id(0)

    @pl.when(k == 0)
    def _init():
        s1_ref[...] = jnp.zeros_like(s1_ref)
        s2_ref[...] = jnp.zeros_like(s2_ref)

    a = a_ref[...]
    wsq1 = w_ref[...] * w_ref[...]

    q_idx = (k - 1) * BQ + jax.lax.broadcasted_iota(jnp.int32, (1, BQ), 1)
    masks = {}
    for d in range(-2, 3):
        t = q_idx - d
        masks[d] = jnp.logical_and(t >= 0, t < t_total).astype(jnp.float32)

    env_top = (masks[-1] * wsq1[128 + 0:128 + 128] + masks[0] * wsq1[384:512]
               + masks[1] * wsq1[640:768] + masks[2] * wsq1[896:1024])
    env_bot = (masks[-2] * wsq1[0:128] + masks[-1] * wsq1[256:384]
               + masks[0] * wsq1[512:640] + masks[1] * wsq1[768:896])
    inv_env = 1.0 / (jnp.concatenate([env_top, env_bot], axis=0) + EPS)

    b = main_ref.shape[0]
    for bi in range(b):
        cur = main_ref[bi].astype(jnp.bfloat16)
        x = jnp.concatenate([s2_ref[bi], s1_ref[bi], cur], axis=1)
        acc = jnp.zeros((HOP, BQ), dtype=jnp.float32)
        for d in (-1, 0, 1):
            xs = x[:, 128 - d:384 - d]
            ad = a[384 + 256 * d:640 + 256 * d, :]
            acc = acc + masks[d] * jax.lax.dot_general(
                ad, xs, (((1,), (0,)), ((), ())),
                preferred_element_type=jnp.float32)
        top = acc[0:128, :] + masks[2] * jax.lax.dot_general(
            a[896:1024, :], x[:, 126:382], (((1,), (0,)), ((), ())),
            preferred_element_type=jnp.float32)
        bot = acc[128:256, :] + masks[-2] * jax.lax.dot_general(
            a[0:128, :], x[:, 130:386], (((1,), (0,)), ((), ())),
            preferred_element_type=jnp.float32)
        y = jnp.concatenate([top, bot], axis=0) * inv_env
        out_ref[bi] = y.T

        s2_ref[bi] = s1_ref[bi, :, BQ - HALO:]
        s1_ref[bi] = cur


@jax.jit
def kernel(spec, window):
    b, nfreq, t = spec.shape
    n_chunks = t // BQ

    a = (jnp.asarray(_idft_matrix())
         * window[:, None]).astype(jnp.bfloat16)
    w2d = window.reshape(WIN, 1)

    out = pl.pallas_call(
        functools.partial(_istft_kernel, t),
        grid=(n_chunks + 1,),
        in_specs=[
            pl.BlockSpec((b, nfreq, BQ),
                         lambda k: (0, 0, jnp.clip(k, 0, t // BQ - 1))),
            pl.BlockSpec((WIN, NFREQ), lambda k: (0, 0)),
            pl.BlockSpec((WIN, 1), lambda k: (0, 0)),
        ],
        out_specs=pl.BlockSpec((b, BQ, HOP),
                               lambda k: (0, jnp.maximum(k - 1, 0), 0)),
        out_shape=jax.ShapeDtypeStruct((b, t, HOP), jnp.float32),
        scratch_shapes=[
            pltpu.VMEM((b, nfreq, BQ), jnp.bfloat16),
            pltpu.VMEM((b, nfreq, HALO), jnp.bfloat16),
        ],
    )(spec, a, w2d)

    return out.reshape(b, t * HOP)


# speedup vs baseline: 1.4665x; 1.0004x over previous
import functools

import jax
import jax.numpy as jnp
import numpy as np
from jax.experimental import pallas as pl
from jax.experimental.pallas import tpu as pltpu

N_FFT = 1024
HOP = 256
WIN = 1024
EPS = 1e-11
NFREQ = N_FFT // 2 + 1
PAD = (WIN - HOP) // 2
BQ = 256
HALO = 128


def _idft_matrix() -> np.ndarray:
    k = np.arange(NFREQ, dtype=np.float64)
    n = np.arange(N_FFT, dtype=np.float64)
    coef = np.full(NFREQ, 2.0)
    coef[0] = 1.0
    coef[NFREQ - 1] = 1.0
    m = (coef[None, :] * np.cos(2.0 * np.pi * np.outer(n, k) / N_FFT)) / N_FFT
    return m.astype(np.float32)


def _istft_kernel(t_total, main_ref, a_ref, w_ref, out_ref, s1_ref, s2_ref):
    k = pl.program_id(0)

    @pl.when(k == 0)
    def _init():
        s1_ref[...] = jnp.zeros_like(s1_ref)
        s2_ref[...] = jnp.zeros_like(s2_ref)

    a = a_ref[...]
    wsq1 = w_ref[...] * w_ref[...]

    q_idx = (k - 1) * BQ + jax.lax.broadcasted_iota(jnp.int32, (1, BQ), 1)
    masks = {}
    for d in range(-2, 3):
        t = q_idx - d
        masks[d] = jnp.logical_and(t >= 0, t < t_total).astype(jnp.float32)

    env_top = (masks[-1] * wsq1[128 + 0:128 + 128] + masks[0] * wsq1[384:512]
               + masks[1] * wsq1[640:768] + masks[2] * wsq1[896:1024])
    env_bot = (masks[-2] * wsq1[0:128] + masks[-1] * wsq1[256:384]
               + masks[0] * wsq1[512:640] + masks[1] * wsq1[768:896])
    inv_env = 1.0 / (jnp.concatenate([env_top, env_bot], axis=0) + EPS)

    b = main_ref.shape[0]
    for bi in range(b):
        cur = main_ref[bi].astype(jnp.bfloat16)
        x = jnp.concatenate([s2_ref[bi], s1_ref[bi], cur], axis=1)
        acc = jnp.zeros((HOP, BQ), dtype=jnp.float32)
        for d in (-1, 0, 1):
            xs = x[:, 128 - d:384 - d]
            ad = a[384 + 256 * d:640 + 256 * d, :]
            acc = acc + masks[d] * jax.lax.dot_general(
                ad, xs, (((1,), (0,)), ((), ())),
                preferred_element_type=jnp.float32)
        top = acc[0:128, :] + masks[2] * jax.lax.dot_general(
            a[896:1024, :], x[:, 126:382], (((1,), (0,)), ((), ())),
            preferred_element_type=jnp.float32)
        bot = acc[128:256, :] + masks[-2] * jax.lax.dot_general(
            a[0:128, :], x[:, 130:386], (((1,), (0,)), ((), ())),
            preferred_element_type=jnp.float32)
        y = jnp.concatenate([top, bot], axis=0) * inv_env
        out_ref[bi] = y.T

        s2_ref[bi] = s1_ref[bi, :, BQ - HALO:]
        s1_ref[bi] = cur


@jax.jit
def kernel(spec, window):
    b, nfreq, t = spec.shape
    n_chunks = t // BQ

    a = (jnp.asarray(_idft_matrix())
         * window[:, None]).astype(jnp.bfloat16)
    w2d = window.reshape(WIN, 1)

    out = pl.pallas_call(
        functools.partial(_istft_kernel, t),
        grid=(n_chunks + 1,),
        in_specs=[
            pl.BlockSpec((b, nfreq, BQ),
                         lambda k: (0, 0, jnp.clip(k, 0, t // BQ - 1))),
            pl.BlockSpec((WIN, NFREQ), lambda k: (0, 0)),
            pl.BlockSpec((WIN, 1), lambda k: (0, 0)),
        ],
        out_specs=pl.BlockSpec((b, BQ, HOP),
                               lambda k: (0, jnp.maximum(k - 1, 0), 0)),
        out_shape=jax.ShapeDtypeStruct((b, t, HOP), jnp.float32),
        scratch_shapes=[
            pltpu.VMEM((b, nfreq, BQ), jnp.bfloat16),
            pltpu.VMEM((b, nfreq, HALO), jnp.bfloat16),
        ],
        compiler_params=pltpu.CompilerParams(
            dimension_semantics=("arbitrary",)),
    )(spec, a, w2d)

    return out.reshape(b, t * HOP)
